# Initial kernel scaffold; baseline (speedup 1.0000x reference)
import jax
import jax.numpy as jnp
from jax.experimental import pallas as pl


def kernel(entity_embed, edge_index, edge_att, W_w, W_b):
    raise NotImplementedError("write your pallas kernel here")


# speedup vs baseline: 4.5188x; 4.5188x over previous
import functools

import jax
import jax.numpy as jnp
from jax import lax
from jax.experimental import pallas as pl
from jax.experimental.pallas import tpu as pltpu
from jax.experimental.pallas import tpu_sc as plsc

N_NODES = 10000
N_EDGES = 320000
D = 128

NC = 2
NS = 16
NW = NC * NS
L = 16

E_W = N_EDGES // NW
E_C = 80
N_CHUNKS = E_W // E_C
R_S = 624
R_REM = N_NODES - NS * R_S
ZB = 104


def _sc_segment_sum(embed, src, dst, att):
    mesh = plsc.VectorSubcoreMesh(core_axis_name="c", subcore_axis_name="s")

    @functools.partial(
        pl.kernel,
        out_type=jax.ShapeDtypeStruct((NC, N_NODES, D), jnp.float32),
        mesh=mesh,
        scratch_types=[
            pltpu.VMEM((E_C,), jnp.int32),
            pltpu.VMEM((E_C,), jnp.int32),
            pltpu.VMEM((E_C,), jnp.float32),
            pltpu.VMEM((E_C, D), jnp.float32),
            pltpu.VMEM((ZB, D), jnp.float32),
            pltpu.VMEM_SHARED((N_NODES, D), jnp.float32),
            pltpu.SemaphoreType.DMA,
        ],
    )
    def k(embed_hbm, src_hbm, dst_hbm, att_hbm, out_hbm,
          src_v, dst_v, att_v, rows_v, zbuf, acc, sem):
        cid = lax.axis_index("c")
        sid = lax.axis_index("s")
        wid = sid * NC + cid

        zero = jnp.zeros((L,), jnp.float32)

        def zero_row(r, _):
            for j in range(D // L):
                zbuf[r, pl.ds(j * L, L)] = zero
            return _

        lax.fori_loop(0, ZB, zero_row, None)
        row0 = sid * R_S
        for t in range(R_S // ZB):
            pltpu.sync_copy(zbuf, acc.at[pl.ds(row0 + t * ZB, ZB)])

        @pl.when(sid == NS - 1)
        def _():
            pltpu.sync_copy(zbuf.at[pl.ds(0, R_REM)],
                            acc.at[pl.ds(NS * R_S, R_REM)])

        plsc.subcore_barrier()

        def chunk(c, _):
            base = wid * E_W + c * E_C
            pltpu.sync_copy(src_hbm.at[pl.ds(base, E_C)], src_v)
            pltpu.sync_copy(dst_hbm.at[pl.ds(base, E_C)], dst_v)
            pltpu.sync_copy(att_hbm.at[pl.ds(base, E_C)], att_v)
            pltpu.async_copy(embed_hbm.at[src_v], rows_v, sem).wait()

            def scale_block(k, _):
                att16 = att_v[pl.ds(k * L, L)]
                for l in range(L):
                    a = att16[l]
                    e = k * L + l
                    for j in range(D // L):
                        sl = pl.ds(j * L, L)
                        rows_v[e, sl] = rows_v[e, sl] * a
                return _

            lax.fori_loop(0, E_C // L, scale_block, None)
            pltpu.sync_copy(rows_v, acc.at[dst_v], add=True)
            return _

        lax.fori_loop(0, N_CHUNKS, chunk, None)
        plsc.subcore_barrier()
        pltpu.sync_copy(acc.at[pl.ds(row0, R_S)],
                        out_hbm.at[cid, pl.ds(row0, R_S)])

        @pl.when(sid == NS - 1)
        def _():
            pltpu.sync_copy(acc.at[pl.ds(NS * R_S, R_REM)],
                            out_hbm.at[cid, pl.ds(NS * R_S, R_REM)])

    return k(embed, src, dst, att)


def _tc_tail_body(e_ref, p_ref, w_ref, b_ref, o_ref):
    h = e_ref[...] + p_ref[0] + p_ref[1]
    y = lax.dot_general(h, w_ref[...], (((1,), (1,)), ((), ())),
                        preferred_element_type=jnp.float32)
    y = y + b_ref[...]
    o_ref[...] = jnp.where(y >= 0, y, 0.01 * y)


def _tc_tail(embed, partials, W_w, W_b):
    BR = 1000
    grid = N_NODES // BR
    return pl.pallas_call(
        _tc_tail_body,
        grid=(grid,),
        in_specs=[
            pl.BlockSpec((BR, D), lambda i: (i, 0)),
            pl.BlockSpec((NC, BR, D), lambda i: (0, i, 0)),
            pl.BlockSpec((D, D), lambda i: (0, 0)),
            pl.BlockSpec((1, D), lambda i: (0, 0)),
        ],
        out_specs=pl.BlockSpec((BR, D), lambda i: (i, 0)),
        out_shape=jax.ShapeDtypeStruct((N_NODES, D), jnp.float32),
    )(embed, partials, W_w, W_b)


@jax.jit
def kernel(entity_embed, edge_index, edge_att, W_w, W_b):
    src = edge_index[0].astype(jnp.int32)
    dst = edge_index[1].astype(jnp.int32)
    att = edge_att.astype(jnp.float32)
    partials = _sc_segment_sum(entity_embed, src, dst, att)
    return _tc_tail(entity_embed, partials, W_w, W_b.reshape(1, D))


# speedup vs baseline: 10.3617x; 2.2930x over previous
import functools

import jax
import jax.numpy as jnp
from jax import lax
from jax.experimental import pallas as pl
from jax.experimental.pallas import tpu as pltpu
from jax.experimental.pallas import tpu_sc as plsc

N_NODES = 10000
N_EDGES = 320000
D = 128

NC = 2
NS = 16
NW = NC * NS
L = 16

E_W = N_EDGES // NW
E_C = 80
N_CHUNKS = E_W // E_C
N_PAIRS = (N_CHUNKS - 1) // 2
R_S = 624
R_REM = N_NODES - NS * R_S
ZB = 104


def _sc_segment_sum(embed, src2, dst2, att2):
    mesh = plsc.VectorSubcoreMesh(core_axis_name="c", subcore_axis_name="s")

    @functools.partial(
        pl.kernel,
        out_type=jax.ShapeDtypeStruct((NC, N_NODES, D), jnp.float32),
        mesh=mesh,
        scratch_types=[
            pltpu.VMEM((N_CHUNKS, E_C), jnp.int32),
            pltpu.VMEM((2, 1, E_C), jnp.int32),
            pltpu.VMEM((2, 1, E_C), jnp.float32),
            pltpu.VMEM((2, E_C, D), jnp.float32),
            pltpu.VMEM_SHARED((N_NODES, D), jnp.float32),
            pltpu.SemaphoreType.DMA,
            pltpu.SemaphoreType.DMA,
        ],
    )
    def k(embed_hbm, src_hbm, dst_hbm, att_hbm, out_hbm,
          src_v, dst_v, att_v, rows_v, acc, sem0, sem1):
        cid = lax.axis_index("c")
        sid = lax.axis_index("s")
        wid = sid * NC + cid

        zero = jnp.zeros((L,), jnp.float32)

        def zero_row(r, _):
            for j in range(D // L):
                rows_v[0, r, pl.ds(j * L, L)] = zero
            return _

        lax.fori_loop(0, E_C, zero_row, None)
        row0 = sid * R_S
        for t in range(R_S // E_C):
            pltpu.sync_copy(rows_v.at[0], acc.at[pl.ds(row0 + t * E_C, E_C)])
        rem = R_S - (R_S // E_C) * E_C
        if rem:
            pltpu.sync_copy(rows_v.at[0, pl.ds(0, rem)],
                            acc.at[pl.ds(row0 + (R_S // E_C) * E_C, rem)])

        @pl.when(sid == NS - 1)
        def _():
            pltpu.sync_copy(rows_v.at[0, pl.ds(0, R_REM)],
                            acc.at[pl.ds(NS * R_S, R_REM)])

        pltpu.sync_copy(src_hbm.at[wid], src_v)
        plsc.subcore_barrier()

        def scale(buf, c):
            def scale_block(kk, _):
                att16 = att_v[buf, 0, pl.ds(kk * L, L)]
                for l in range(L):
                    a = att16[l]
                    for j in range(D // L):
                        sl = pl.ds(j * L, L)
                        rows_v[buf, kk * L + l, sl] = rows_v[buf, kk * L + l, sl] * a
                return _

            lax.fori_loop(0, E_C // L, scale_block, None)

        def gather(c, buf, sem):
            pltpu.async_copy(dst_hbm.at[wid, c], dst_v.at[buf], sem)
            pltpu.async_copy(att_hbm.at[wid, c], att_v.at[buf], sem)
            pltpu.async_copy(embed_hbm.at[src_v.at[c]], rows_v.at[buf], sem)

        def drain(c, buf, sem):
            pltpu.make_async_copy(dst_hbm.at[wid, c], dst_v.at[buf], sem).wait()
            pltpu.make_async_copy(att_hbm.at[wid, c], att_v.at[buf], sem).wait()
            pltpu.make_async_copy(embed_hbm.at[src_v.at[c]], rows_v.at[buf], sem).wait()
            scale(buf, c)
            pltpu.sync_copy(rows_v.at[buf], acc.at[dst_v.at[buf, 0]], add=True)

        gather(0, 0, sem0)

        def pair(t, _):
            a = 2 * t
            gather(a + 1, 1, sem1)
            drain(a, 0, sem0)
            gather(a + 2, 0, sem0)
            drain(a + 1, 1, sem1)
            return _

        lax.fori_loop(0, N_PAIRS, pair, None)
        drain(N_CHUNKS - 1, 0, sem0)

        plsc.subcore_barrier()
        pltpu.sync_copy(acc.at[pl.ds(row0, R_S)],
                        out_hbm.at[cid, pl.ds(row0, R_S)])

        @pl.when(sid == NS - 1)
        def _():
            pltpu.sync_copy(acc.at[pl.ds(NS * R_S, R_REM)],
                            out_hbm.at[cid, pl.ds(NS * R_S, R_REM)])

    return k(embed, src2, dst2, att2)


def _tc_tail_body(e_ref, p_ref, w_ref, b_ref, o_ref):
    h = e_ref[...] + p_ref[0] + p_ref[1]
    y = lax.dot_general(h, w_ref[...], (((1,), (1,)), ((), ())),
                        preferred_element_type=jnp.float32)
    y = y + b_ref[...]
    o_ref[...] = jnp.where(y >= 0, y, 0.01 * y)


def _tc_tail(embed, partials, W_w, W_b):
    BR = 1000
    grid = N_NODES // BR
    return pl.pallas_call(
        _tc_tail_body,
        grid=(grid,),
        in_specs=[
            pl.BlockSpec((BR, D), lambda i: (i, 0)),
            pl.BlockSpec((NC, BR, D), lambda i: (0, i, 0)),
            pl.BlockSpec((D, D), lambda i: (0, 0)),
            pl.BlockSpec((1, D), lambda i: (0, 0)),
        ],
        out_specs=pl.BlockSpec((BR, D), lambda i: (i, 0)),
        out_shape=jax.ShapeDtypeStruct((N_NODES, D), jnp.float32),
    )(embed, partials, W_w, W_b)


@jax.jit
def kernel(entity_embed, edge_index, edge_att, W_w, W_b):
    src = edge_index[0].astype(jnp.int32).reshape(NW, N_CHUNKS, E_C)
    dst = edge_index[1].astype(jnp.int32).reshape(NW, N_CHUNKS, 1, E_C)
    att = edge_att.astype(jnp.float32).reshape(NW, N_CHUNKS, 1, E_C)
    partials = _sc_segment_sum(entity_embed, src, dst, att)
    return _tc_tail(entity_embed, partials, W_w, W_b.reshape(1, D))


# speedup vs baseline: 11.3681x; 1.0971x over previous
import functools

import jax
import jax.numpy as jnp
from jax import lax
from jax.experimental import pallas as pl
from jax.experimental.pallas import tpu as pltpu
from jax.experimental.pallas import tpu_sc as plsc

N_NODES = 10000
N_EDGES = 320000
D = 128

NC = 2
NS = 16
NW = NC * NS
L = 16

E_W = N_EDGES // NW
E_C = 80
N_CHUNKS = E_W // E_C
NB = 3
N_STEADY = (N_CHUNKS - 5) // NB
R_S = 624
R_REM = N_NODES - NS * R_S


def _sc_segment_sum(embed, src2, dst2, att2):
    mesh = plsc.VectorSubcoreMesh(core_axis_name="c", subcore_axis_name="s")

    @functools.partial(
        pl.kernel,
        out_type=jax.ShapeDtypeStruct((NC, N_NODES, D), jnp.float32),
        mesh=mesh,
        scratch_types=[
            pltpu.VMEM((N_CHUNKS, E_C), jnp.int32),
            pltpu.VMEM((NB, 1, E_C), jnp.int32),
            pltpu.VMEM((NB, 1, E_C), jnp.float32),
            pltpu.VMEM((NB, E_C, D), jnp.float32),
            pltpu.VMEM_SHARED((N_NODES, D), jnp.float32),
            [pltpu.SemaphoreType.DMA] * NB,
            [pltpu.SemaphoreType.DMA] * NB,
        ],
    )
    def k(embed_hbm, src_hbm, dst_hbm, att_hbm, out_hbm,
          src_v, dst_v, att_v, rows_v, acc, gsems, ssems):
        cid = lax.axis_index("c")
        sid = lax.axis_index("s")
        wid = sid * NC + cid

        zero = jnp.zeros((L,), jnp.float32)

        def zero_row(r, _):
            for j in range(D // L):
                rows_v[0, r, pl.ds(j * L, L)] = zero
            return _

        lax.fori_loop(0, E_C, zero_row, None)
        row0 = sid * R_S
        for t in range(R_S // E_C):
            pltpu.sync_copy(rows_v.at[0], acc.at[pl.ds(row0 + t * E_C, E_C)])
        rem = R_S - (R_S // E_C) * E_C
        if rem:
            pltpu.sync_copy(rows_v.at[0, pl.ds(0, rem)],
                            acc.at[pl.ds(row0 + (R_S // E_C) * E_C, rem)])

        @pl.when(sid == NS - 1)
        def _():
            pltpu.sync_copy(rows_v.at[0, pl.ds(0, R_REM)],
                            acc.at[pl.ds(NS * R_S, R_REM)])

        pltpu.sync_copy(src_hbm.at[wid], src_v)
        plsc.subcore_barrier()

        def scale(b, c):
            def scale_block(kk, _):
                att16 = att_v[b, 0, pl.ds(kk * L, L)]
                for l in range(L):
                    a = att16[l]
                    for j in range(D // L):
                        sl = pl.ds(j * L, L)
                        rows_v[b, kk * L + l, sl] = rows_v[b, kk * L + l, sl] * a
                return _

            lax.fori_loop(0, E_C // L, scale_block, None)

        def gather(c, b):
            pltpu.async_copy(dst_hbm.at[wid, c], dst_v.at[b], gsems[b])
            pltpu.async_copy(att_hbm.at[wid, c], att_v.at[b], gsems[b])
            pltpu.async_copy(embed_hbm.at[src_v.at[c]], rows_v.at[b], gsems[b])

        def wait_gather(c, b):
            pltpu.make_async_copy(dst_hbm.at[wid, c], dst_v.at[b], gsems[b]).wait()
            pltpu.make_async_copy(att_hbm.at[wid, c], att_v.at[b], gsems[b]).wait()
            pltpu.make_async_copy(embed_hbm.at[src_v.at[c]], rows_v.at[b],
                                  gsems[b]).wait()

        def scatter(b):
            pltpu.async_copy(rows_v.at[b], acc.at[dst_v.at[b, 0]], ssems[b],
                             add=True)

        def wait_scatter(b):
            pltpu.make_async_copy(rows_v.at[b], acc.at[dst_v.at[b, 0]],
                                  ssems[b]).wait()

        def proc(c, b, nxt, wait_prev_scatter, issue_next):
            wait_gather(c, b)
            scale(b, c)
            scatter(b)
            if issue_next:
                if wait_prev_scatter:
                    wait_scatter(nxt)
                gather_c2 = c + 2
                gather(gather_c2, nxt)

        gather(0, 0)
        gather(1, 1)
        proc(0, 0, 2, False, True)
        proc(1, 1, 0, True, True)
        proc(2, 2, 1, True, True)

        def steady(t, _):
            c = NB * t + NB
            for u in range(NB):
                b = u
                nxt = (u + 2) % NB
                cc = c + u
                wait_gather(cc, b)
                scale(b, cc)
                scatter(b)
                wait_scatter(nxt)
                gather(cc + 2, nxt)
            return _

        lax.fori_loop(0, N_STEADY, steady, None)

        proc(N_CHUNKS - 2, (N_CHUNKS - 2) % NB, 0, False, False)
        proc(N_CHUNKS - 1, (N_CHUNKS - 1) % NB, 0, False, False)
        for b in range(NB):
            wait_scatter(b)

        plsc.subcore_barrier()
        pltpu.sync_copy(acc.at[pl.ds(row0, R_S)],
                        out_hbm.at[cid, pl.ds(row0, R_S)])

        @pl.when(sid == NS - 1)
        def _():
            pltpu.sync_copy(acc.at[pl.ds(NS * R_S, R_REM)],
                            out_hbm.at[cid, pl.ds(NS * R_S, R_REM)])

    return k(embed, src2, dst2, att2)


def _tc_tail_body(e_ref, p_ref, w_ref, b_ref, o_ref):
    h = e_ref[...] + p_ref[0] + p_ref[1]
    y = lax.dot_general(h, w_ref[...], (((1,), (1,)), ((), ())),
                        preferred_element_type=jnp.float32)
    y = y + b_ref[...]
    o_ref[...] = jnp.where(y >= 0, y, 0.01 * y)


def _tc_tail(embed, partials, W_w, W_b):
    BR = 1000
    grid = N_NODES // BR
    return pl.pallas_call(
        _tc_tail_body,
        grid=(grid,),
        in_specs=[
            pl.BlockSpec((BR, D), lambda i: (i, 0)),
            pl.BlockSpec((NC, BR, D), lambda i: (0, i, 0)),
            pl.BlockSpec((D, D), lambda i: (0, 0)),
            pl.BlockSpec((1, D), lambda i: (0, 0)),
        ],
        out_specs=pl.BlockSpec((BR, D), lambda i: (i, 0)),
        out_shape=jax.ShapeDtypeStruct((N_NODES, D), jnp.float32),
    )(embed, partials, W_w, W_b)


@jax.jit
def kernel(entity_embed, edge_index, edge_att, W_w, W_b):
    src = edge_index[0].astype(jnp.int32).reshape(NW, N_CHUNKS, E_C)
    dst = edge_index[1].astype(jnp.int32).reshape(NW, N_CHUNKS, 1, E_C)
    att = edge_att.astype(jnp.float32).reshape(NW, N_CHUNKS, 1, E_C)
    partials = _sc_segment_sum(entity_embed, src, dst, att)
    return _tc_tail(entity_embed, partials, W_w, W_b.reshape(1, D))


# speedup vs baseline: 12.4578x; 1.0959x over previous
import functools

import jax
import jax.numpy as jnp
from jax import lax
from jax.experimental import pallas as pl
from jax.experimental.pallas import tpu as pltpu
from jax.experimental.pallas import tpu_sc as plsc

N_NODES = 10000
N_EDGES = 320000
D = 128

NC = 2
NS = 16
NW = NC * NS
L = 16

E_W = N_EDGES // NW
E_C = 80
N_CHUNKS = E_W // E_C
NB = 3
N_STEADY = (N_CHUNKS - 5) // NB
R_S = 624
R_REM = N_NODES - NS * R_S


def _sc_segment_sum(embed, src2, dst2, att2):
    mesh = plsc.VectorSubcoreMesh(core_axis_name="c", subcore_axis_name="s")

    @functools.partial(
        pl.kernel,
        out_type=jax.ShapeDtypeStruct((NC, N_NODES, D), jnp.float32),
        mesh=mesh,
        scratch_types=[
            pltpu.VMEM((E_W,), jnp.int32),
            pltpu.VMEM((NB, E_C), jnp.int32),
            pltpu.VMEM((NB, E_C), jnp.float32),
            pltpu.VMEM((NB, E_C, D), jnp.float32),
            pltpu.VMEM_SHARED((N_NODES, D), jnp.float32),
            [pltpu.SemaphoreType.DMA] * NB,
            [pltpu.SemaphoreType.DMA] * NB,
        ],
    )
    def k(embed_hbm, src_hbm, dst_hbm, att_hbm, out_hbm,
          src_v, dst_v, att_v, rows_v, acc, gsems, ssems):
        cid = lax.axis_index("c")
        sid = lax.axis_index("s")
        wid = sid * NC + cid

        zero = jnp.zeros((L,), jnp.float32)

        def zero_row(r, _):
            for j in range(D // L):
                rows_v[0, r, pl.ds(j * L, L)] = zero
            return _

        lax.fori_loop(0, E_C, zero_row, None)
        row0 = sid * R_S
        for t in range(R_S // E_C):
            pltpu.sync_copy(rows_v.at[0], acc.at[pl.ds(row0 + t * E_C, E_C)])
        rem = R_S - (R_S // E_C) * E_C
        if rem:
            pltpu.sync_copy(rows_v.at[0, pl.ds(0, rem)],
                            acc.at[pl.ds(row0 + (R_S // E_C) * E_C, rem)])

        @pl.when(sid == NS - 1)
        def _():
            pltpu.sync_copy(rows_v.at[0, pl.ds(0, R_REM)],
                            acc.at[pl.ds(NS * R_S, R_REM)])

        pltpu.sync_copy(src_hbm.at[pl.ds(wid * E_W, E_W)], src_v)
        plsc.subcore_barrier()

        def scale(b, c):
            def scale_block(kk, _):
                att16 = att_v[b, pl.ds(kk * L, L)]
                for l in range(L):
                    a = att16[l]
                    for j in range(D // L):
                        sl = pl.ds(j * L, L)
                        rows_v[b, kk * L + l, sl] = rows_v[b, kk * L + l, sl] * a
                return _

            lax.fori_loop(0, E_C // L, scale_block, None)

        def gather(c, b):
            base = wid * E_W + c * E_C
            pltpu.async_copy(dst_hbm.at[pl.ds(base, E_C)], dst_v.at[b], gsems[b])
            pltpu.async_copy(att_hbm.at[pl.ds(base, E_C)], att_v.at[b], gsems[b])
            pltpu.async_copy(embed_hbm.at[src_v.at[pl.ds(c * E_C, E_C)]],
                             rows_v.at[b], gsems[b])

        def wait_gather(c, b):
            base = wid * E_W + c * E_C
            pltpu.make_async_copy(dst_hbm.at[pl.ds(base, E_C)], dst_v.at[b],
                                  gsems[b]).wait()
            pltpu.make_async_copy(att_hbm.at[pl.ds(base, E_C)], att_v.at[b],
                                  gsems[b]).wait()
            pltpu.make_async_copy(embed_hbm.at[src_v.at[pl.ds(c * E_C, E_C)]],
                                  rows_v.at[b], gsems[b]).wait()

        def scatter(b):
            pltpu.async_copy(rows_v.at[b], acc.at[dst_v.at[b]], ssems[b],
                             add=True)

        def wait_scatter(b):
            pltpu.make_async_copy(rows_v.at[b], acc.at[dst_v.at[b]],
                                  ssems[b]).wait()

        def proc(c, b, nxt, wait_prev_scatter, issue_next):
            wait_gather(c, b)
            scale(b, c)
            scatter(b)
            if issue_next:
                if wait_prev_scatter:
                    wait_scatter(nxt)
                gather_c2 = c + 2
                gather(gather_c2, nxt)

        gather(0, 0)
        gather(1, 1)
        proc(0, 0, 2, False, True)
        proc(1, 1, 0, True, True)
        proc(2, 2, 1, True, True)

        def steady(t, _):
            c = NB * t + NB
            for u in range(NB):
                b = u
                nxt = (u + 2) % NB
                cc = c + u
                wait_gather(cc, b)
                scale(b, cc)
                scatter(b)
                wait_scatter(nxt)
                gather(cc + 2, nxt)
            return _

        lax.fori_loop(0, N_STEADY, steady, None)

        proc(N_CHUNKS - 2, (N_CHUNKS - 2) % NB, 0, False, False)
        proc(N_CHUNKS - 1, (N_CHUNKS - 1) % NB, 0, False, False)
        for b in range(NB):
            wait_scatter(b)

        plsc.subcore_barrier()
        pltpu.sync_copy(acc.at[pl.ds(row0, R_S)],
                        out_hbm.at[cid, pl.ds(row0, R_S)])

        @pl.when(sid == NS - 1)
        def _():
            pltpu.sync_copy(acc.at[pl.ds(NS * R_S, R_REM)],
                            out_hbm.at[cid, pl.ds(NS * R_S, R_REM)])

    return k(embed, src2, dst2, att2)


def _tc_tail_body(e_ref, p_ref, w_ref, b_ref, o_ref):
    h = e_ref[...] + p_ref[0] + p_ref[1]
    y = lax.dot_general(h, w_ref[...], (((1,), (1,)), ((), ())),
                        preferred_element_type=jnp.float32)
    y = y + b_ref[...]
    o_ref[...] = jnp.where(y >= 0, y, 0.01 * y)


def _tc_tail(embed, partials, W_w, W_b):
    BR = 1000
    grid = N_NODES // BR
    return pl.pallas_call(
        _tc_tail_body,
        grid=(grid,),
        in_specs=[
            pl.BlockSpec((BR, D), lambda i: (i, 0)),
            pl.BlockSpec((NC, BR, D), lambda i: (0, i, 0)),
            pl.BlockSpec((D, D), lambda i: (0, 0)),
            pl.BlockSpec((1, D), lambda i: (0, 0)),
        ],
        out_specs=pl.BlockSpec((BR, D), lambda i: (i, 0)),
        out_shape=jax.ShapeDtypeStruct((N_NODES, D), jnp.float32),
    )(embed, partials, W_w, W_b)


@jax.jit
def kernel(entity_embed, edge_index, edge_att, W_w, W_b):
    src = edge_index[0].astype(jnp.int32)
    dst = edge_index[1].astype(jnp.int32)
    att = edge_att
    partials = _sc_segment_sum(entity_embed, src, dst, att)
    return _tc_tail(entity_embed, partials, W_w, W_b.reshape(1, D))
